# baseline (speedup 1.0000x reference)
import jax
import jax.numpy as jnp
from jax import lax
from jax.experimental import pallas as pl
from jax.experimental.pallas import tpu as pltpu
from jax.experimental.pallas import tpu_sc as plsc

M = 1_000_000
D = 64
NC = 2
NS = 16
NW = NC * NS
W = 768
S = 196608
CS = S // W
ITERS = CS // NW
G = 8
PASSES = W // (16 * G)
WT = 32768


def _sc_body(items_hbm, u_hbm, out_hbm, in_buf0, in_buf1, out_buf0, out_buf1,
             u_vmem, sem0, sem1, osem0, osem1):
    wid = lax.axis_index("s") * NC + lax.axis_index("c")
    in_bufs = (in_buf0, in_buf1)
    sems = (sem0, sem1)
    out_bufs = (out_buf0, out_buf1)
    osems = (osem0, osem1)

    pltpu.sync_copy(u_hbm, u_vmem)

    def start_in(j, b):
        col0 = (wid + NW * j) * W
        pltpu.async_copy(items_hbm.at[:, pl.ds(col0, W)], in_bufs[b], sems[b])

    def wait_in(j, b):
        col0 = (wid + NW * j) * W
        pltpu.make_async_copy(items_hbm.at[:, pl.ds(col0, W)], in_bufs[b],
                              sems[b]).wait()

    def compute(j, b):
        buf = in_bufs[b]
        out_buf = out_bufs[b]

        def one_pass(p, _):
            base = p * (16 * G)

            def d_block(db, accs):
                accs = list(accs)
                for k in range(8):
                    d = db * 8 + k
                    u_d = u_vmem[d, :]
                    for g in range(G):
                        v = buf[d, pl.ds(base + g * 16, 16)]
                        accs[g] = accs[g] + v * u_d
                return tuple(accs)

            accs = lax.fori_loop(
                0, D // 8, d_block,
                tuple(jnp.zeros((16,), jnp.float32) for _ in range(G)))
            for g in range(G):
                out_buf[pl.ds(base + g * 16, 16)] = accs[g]
            return 0

        lax.fori_loop(0, PASSES, one_pass, 0)
        pltpu.async_copy(out_buf, out_hbm.at[pl.ds((wid + NW * j) * W, W)],
                         osems[b])

    start_in(0, 0)

    def step(jp, _):
        for b in (0, 1):
            j = 2 * jp + b

            @pl.when(j + 1 < ITERS)
            def _():
                start_in(j + 1, 1 - b)

            wait_in(j, b)

            @pl.when(j - 2 >= 0)
            def _():
                pltpu.make_async_copy(
                    out_bufs[b],
                    out_hbm.at[pl.ds((wid + NW * (j - 2)) * W, W)],
                    osems[b]).wait()

            compute(j, b)
        return 0

    lax.fori_loop(0, ITERS // 2, step, 0)

    for j in (ITERS - 2, ITERS - 1):
        pltpu.make_async_copy(
            out_bufs[j % 2],
            out_hbm.at[pl.ds((wid + NW * j) * W, W)],
            osems[j % 2]).wait()


def _sc_matvec(items_t, u_b):
    mesh = plsc.VectorSubcoreMesh(core_axis_name="c", subcore_axis_name="s")
    f = pl.kernel(
        _sc_body,
        out_type=jax.ShapeDtypeStruct((S,), jnp.float32),
        mesh=mesh,
        scratch_types=[
            pltpu.VMEM((D, W), jnp.float32),
            pltpu.VMEM((D, W), jnp.float32),
            pltpu.VMEM((W,), jnp.float32),
            pltpu.VMEM((W,), jnp.float32),
            pltpu.VMEM((D, 16), jnp.float32),
            pltpu.SemaphoreType.DMA,
            pltpu.SemaphoreType.DMA,
            pltpu.SemaphoreType.DMA,
            pltpu.SemaphoreType.DMA,
        ],
        compiler_params=pltpu.CompilerParams(needs_layout_passes=False,
                                             use_tc_tiling_on_sc=True),
    )
    return f(items_t, u_b)


def _tc_body(u_ref, x_ref, o_ref):
    o_ref[...] = jnp.sum(x_ref[...] * u_ref[...], axis=0)


def _tc_matvec(items_t, u_col):
    n = M - S
    grid = ((n + WT - 1) // WT,)
    return pl.pallas_call(
        _tc_body,
        grid=grid,
        in_specs=[
            pl.BlockSpec((D, 1), lambda i: (0, 0)),
            pl.BlockSpec((D, WT), lambda i: (0, i + S // WT)),
        ],
        out_specs=pl.BlockSpec((WT,), lambda i: (i,)),
        out_shape=jax.ShapeDtypeStruct((n,), jnp.float32),
    )(u_col, items_t)


@jax.jit
def _hybrid(items_t, u_b, u_col):
    sc = _sc_matvec(items_t, u_b)
    tc = _tc_matvec(items_t, u_col)
    return jnp.concatenate([sc, tc])


def kernel(items_emb, user_emb):
    u_b = jnp.broadcast_to(user_emb.reshape(D, 1), (D, 16))
    return _hybrid(items_emb.T, u_b, user_emb.reshape(D, 1))


# speedup vs baseline: 1.0105x; 1.0105x over previous
import jax
import jax.numpy as jnp
from jax import lax
from jax.experimental import pallas as pl
from jax.experimental.pallas import tpu as pltpu
from jax.experimental.pallas import tpu_sc as plsc

M = 1_000_000
D = 64
NC = 2
NS = 16
NW = NC * NS
W = 768
S = 98304
CS = S // W
ITERS = CS // NW
G = 8
PASSES = W // (16 * G)
WT = 32768


def _sc_body(items_hbm, u_hbm, out_hbm, in_buf0, in_buf1, out_buf0, out_buf1,
             u_vmem, sem0, sem1, osem0, osem1):
    wid = lax.axis_index("s") * NC + lax.axis_index("c")
    in_bufs = (in_buf0, in_buf1)
    sems = (sem0, sem1)
    out_bufs = (out_buf0, out_buf1)
    osems = (osem0, osem1)

    pltpu.sync_copy(u_hbm, u_vmem)

    def start_in(j, b):
        col0 = (wid + NW * j) * W
        pltpu.async_copy(items_hbm.at[:, pl.ds(col0, W)], in_bufs[b], sems[b])

    def wait_in(j, b):
        col0 = (wid + NW * j) * W
        pltpu.make_async_copy(items_hbm.at[:, pl.ds(col0, W)], in_bufs[b],
                              sems[b]).wait()

    def compute(j, b):
        buf = in_bufs[b]
        out_buf = out_bufs[b]

        def one_pass(p, _):
            base = p * (16 * G)

            def d_block(db, accs):
                accs = list(accs)
                for k in range(8):
                    d = db * 8 + k
                    u_d = u_vmem[d, :]
                    for g in range(G):
                        v = buf[d, pl.ds(base + g * 16, 16)]
                        accs[g] = accs[g] + v * u_d
                return tuple(accs)

            accs = lax.fori_loop(
                0, D // 8, d_block,
                tuple(jnp.zeros((16,), jnp.float32) for _ in range(G)))
            for g in range(G):
                out_buf[pl.ds(base + g * 16, 16)] = accs[g]
            return 0

        lax.fori_loop(0, PASSES, one_pass, 0)
        pltpu.async_copy(out_buf, out_hbm.at[pl.ds((wid + NW * j) * W, W)],
                         osems[b])

    start_in(0, 0)

    def step(jp, _):
        for b in (0, 1):
            j = 2 * jp + b

            @pl.when(j + 1 < ITERS)
            def _():
                start_in(j + 1, 1 - b)

            wait_in(j, b)

            @pl.when(j - 2 >= 0)
            def _():
                pltpu.make_async_copy(
                    out_bufs[b],
                    out_hbm.at[pl.ds((wid + NW * (j - 2)) * W, W)],
                    osems[b]).wait()

            compute(j, b)
        return 0

    lax.fori_loop(0, ITERS // 2, step, 0)

    for j in (ITERS - 2, ITERS - 1):
        pltpu.make_async_copy(
            out_bufs[j % 2],
            out_hbm.at[pl.ds((wid + NW * j) * W, W)],
            osems[j % 2]).wait()


def _sc_matvec(items_t, u_b):
    mesh = plsc.VectorSubcoreMesh(core_axis_name="c", subcore_axis_name="s")
    f = pl.kernel(
        _sc_body,
        out_type=jax.ShapeDtypeStruct((S,), jnp.float32),
        mesh=mesh,
        scratch_types=[
            pltpu.VMEM((D, W), jnp.float32),
            pltpu.VMEM((D, W), jnp.float32),
            pltpu.VMEM((W,), jnp.float32),
            pltpu.VMEM((W,), jnp.float32),
            pltpu.VMEM((D, 16), jnp.float32),
            pltpu.SemaphoreType.DMA,
            pltpu.SemaphoreType.DMA,
            pltpu.SemaphoreType.DMA,
            pltpu.SemaphoreType.DMA,
        ],
        compiler_params=pltpu.CompilerParams(needs_layout_passes=False,
                                             use_tc_tiling_on_sc=True),
    )
    return f(items_t, u_b)


def _tc_body(u_ref, x_ref, o_ref):
    o_ref[...] = jnp.sum(x_ref[...] * u_ref[...], axis=0)


def _tc_matvec(items_t, u_col):
    n = M - S
    grid = ((n + WT - 1) // WT,)
    return pl.pallas_call(
        _tc_body,
        grid=grid,
        in_specs=[
            pl.BlockSpec((D, 1), lambda i: (0, 0)),
            pl.BlockSpec((D, WT), lambda i: (0, i + S // WT)),
        ],
        out_specs=pl.BlockSpec((WT,), lambda i: (i,)),
        out_shape=jax.ShapeDtypeStruct((n,), jnp.float32),
    )(u_col, items_t)


@jax.jit
def _hybrid(items_t, u_b, u_col):
    sc = _sc_matvec(items_t, u_b)
    tc = _tc_matvec(items_t, u_col)
    return jnp.concatenate([sc, tc])


def kernel(items_emb, user_emb):
    u_b = jnp.broadcast_to(user_emb.reshape(D, 1), (D, 16))
    return _hybrid(items_emb.T, u_b, user_emb.reshape(D, 1))
